# Initial kernel scaffold; baseline (speedup 1.0000x reference)
import jax
import jax.numpy as jnp
from jax.experimental import pallas as pl


def kernel(x, edge_index, batch, gcn_w0, gcn_b0, gcn_w1, gcn_b1, gcn_w2, gcn_b2, gcn_w3, gcn_b3, attn_in_w, attn_in_b, attn_out_w, attn_out_b, fc_w0, fc_b0, fc_w1, fc_b1, fc_w2, fc_b2, void_w0, void_b0, void_w1, void_b1):
    raise NotImplementedError("write your pallas kernel here")


# speedup vs baseline: 17.2925x; 17.2925x over previous
import functools

import jax
import jax.numpy as jnp
from jax import lax
from jax.experimental import pallas as pl
from jax.experimental.pallas import tpu as pltpu
from jax.experimental.pallas import tpu_sc as plsc

N = 10000
D = 128
E = 320000
NC = 2
NS = 16
NW = NC * NS
EPT = E // NW
CHUNK = 125
NCH = EPT // CHUNK
RPT = 640
RPT_LAST = N - RPT * (NS - 1)


def _deg_body(dst_hbm, z16_hbm, out_hbm, idx_v, ones_v, tab_sh):
    c = lax.axis_index("c")
    s = lax.axis_index("s")
    one16 = jnp.ones((16,), jnp.float32)

    def fill_ones(i, carry):
        ones_v[i, :] = one16
        return carry

    lax.fori_loop(0, CHUNK, fill_ones, 0)

    @pl.when(s < NS - 1)
    def _():
        pltpu.sync_copy(z16_hbm.at[pl.ds(s * RPT, RPT)],
                        tab_sh.at[pl.ds(s * RPT, RPT)])

    @pl.when(s == NS - 1)
    def _():
        pltpu.sync_copy(z16_hbm.at[pl.ds(N - RPT_LAST, RPT_LAST)],
                        tab_sh.at[pl.ds(N - RPT_LAST, RPT_LAST)])

    plsc.subcore_barrier()

    tb = (c * NS + s) * NCH
    pltpu.sync_copy(dst_hbm.at[pl.ds(tb, NCH)], idx_v)

    def step(j, carry):
        pltpu.sync_copy(ones_v, tab_sh.at[idx_v.at[j]], add=True)
        return carry

    lax.fori_loop(0, NCH, step, 0)
    plsc.subcore_barrier()

    @pl.when(s < NS - 1)
    def _():
        pltpu.sync_copy(tab_sh.at[pl.ds(s * RPT, RPT)],
                        out_hbm.at[c, pl.ds(s * RPT, RPT)])

    @pl.when(s == NS - 1)
    def _():
        pltpu.sync_copy(tab_sh.at[pl.ds(N - RPT_LAST, RPT_LAST)],
                        out_hbm.at[c, pl.ds(N - RPT_LAST, RPT_LAST)])


@functools.cache
def _deg_call():
    mesh = plsc.VectorSubcoreMesh(
        core_axis_name="c", subcore_axis_name="s",
        num_cores=NC, num_subcores=NS)
    return pl.kernel(
        _deg_body,
        out_type=jax.ShapeDtypeStruct((NC, N, 16), jnp.float32),
        mesh=mesh,
        scratch_types=[
            pltpu.VMEM((NCH, CHUNK), jnp.int32),
            pltpu.VMEM((CHUNK, 16), jnp.float32),
            pltpu.VMEM_SHARED((N, 16), jnp.float32),
        ],
    )


def _edge_body(g_hbm, src_hbm, dst_hbm, z128_hbm, out_hbm,
               srcs_v, dsts_v, rows_v, acc_sh, sem):
    c = lax.axis_index("c")
    s = lax.axis_index("s")

    @pl.when(s < NS - 1)
    def _():
        pltpu.sync_copy(z128_hbm.at[pl.ds(s * RPT, RPT)],
                        acc_sh.at[pl.ds(s * RPT, RPT)])

    @pl.when(s == NS - 1)
    def _():
        pltpu.sync_copy(z128_hbm.at[pl.ds(N - RPT_LAST, RPT_LAST)],
                        acc_sh.at[pl.ds(N - RPT_LAST, RPT_LAST)])

    plsc.subcore_barrier()

    tb = (c * NS + s) * NCH
    pltpu.sync_copy(src_hbm.at[pl.ds(tb, NCH)], srcs_v)
    pltpu.sync_copy(dst_hbm.at[pl.ds(tb, NCH)], dsts_v)

    def step(j, carry):
        pltpu.async_copy(g_hbm.at[srcs_v.at[j]], rows_v, sem).wait()
        pltpu.sync_copy(rows_v, acc_sh.at[dsts_v.at[j]], add=True)
        return carry

    lax.fori_loop(0, NCH, step, 0)
    plsc.subcore_barrier()

    @pl.when(s < NS - 1)
    def _():
        pltpu.sync_copy(acc_sh.at[pl.ds(s * RPT, RPT)],
                        out_hbm.at[c, pl.ds(s * RPT, RPT)])

    @pl.when(s == NS - 1)
    def _():
        pltpu.sync_copy(acc_sh.at[pl.ds(N - RPT_LAST, RPT_LAST)],
                        out_hbm.at[c, pl.ds(N - RPT_LAST, RPT_LAST)])


@functools.cache
def _edge_call():
    mesh = plsc.VectorSubcoreMesh(
        core_axis_name="c", subcore_axis_name="s",
        num_cores=NC, num_subcores=NS)
    return pl.kernel(
        _edge_body,
        out_type=jax.ShapeDtypeStruct((NC, N, D), jnp.float32),
        mesh=mesh,
        scratch_types=[
            pltpu.VMEM((NCH, CHUNK), jnp.int32),
            pltpu.VMEM((NCH, CHUNK), jnp.int32),
            pltpu.VMEM((CHUNK, D), jnp.float32),
            pltpu.VMEM_SHARED((N, D), jnp.float32),
            pltpu.SemaphoreType.DMA,
        ],
    )


B = 2000
G = N // B


def _init_tc(deg_ref, x_ref, w0t_ref, g0_ref, dinv_ref):
    deg = deg_ref[0, :, 0] + deg_ref[1, :, 0] + 1.0
    dinv = lax.rsqrt(deg)
    h = jnp.dot(x_ref[...], w0t_ref[...], preferred_element_type=jnp.float32)
    g0_ref[...] = h * dinv[:, None]
    dinv_ref[...] = dinv[:, None]


_init_call = pl.pallas_call(
    _init_tc,
    grid=(G,),
    in_specs=[
        pl.BlockSpec((NC, B, 16), lambda i: (0, i, 0)),
        pl.BlockSpec((B, D), lambda i: (i, 0)),
        pl.BlockSpec((D, D), lambda i: (0, 0)),
    ],
    out_specs=[
        pl.BlockSpec((B, D), lambda i: (i, 0)),
        pl.BlockSpec((B, 1), lambda i: (i, 0)),
    ],
    out_shape=[
        jax.ShapeDtypeStruct((N, D), jnp.float32),
        jax.ShapeDtypeStruct((N, 1), jnp.float32),
    ],
)


def _mid_tc(acc_ref, g_ref, dinv_ref, b_ref, wt_ref, out_ref):
    dinv = dinv_ref[...]
    t = (acc_ref[0] + acc_ref[1] + g_ref[...]) * dinv + b_ref[...]
    t = jnp.maximum(t, 0.0)
    out_ref[...] = jnp.dot(
        t, wt_ref[...], preferred_element_type=jnp.float32) * dinv


_mid_call = pl.pallas_call(
    _mid_tc,
    grid=(G,),
    in_specs=[
        pl.BlockSpec((NC, B, D), lambda i: (0, i, 0)),
        pl.BlockSpec((B, D), lambda i: (i, 0)),
        pl.BlockSpec((B, 1), lambda i: (i, 0)),
        pl.BlockSpec((1, D), lambda i: (0, 0)),
        pl.BlockSpec((D, D), lambda i: (0, 0)),
    ],
    out_specs=pl.BlockSpec((B, D), lambda i: (i, 0)),
    out_shape=jax.ShapeDtypeStruct((N, D), jnp.float32),
)


def _final_tc(acc_ref, g_ref, dinv_ref, b3_ref, wvt_ref, bv_ref, wot_ref,
              bo_ref, f0t_ref, fb0_ref, f1t_ref, fb1_ref, f2t_ref, fb2_ref,
              v0t_ref, vb0_ref, v1t_ref, vb1_ref, w_ref, vd_ref):
    dinv = dinv_ref[...]
    h4 = (acc_ref[0] + acc_ref[1] + g_ref[...]) * dinv + b3_ref[...]
    dot = functools.partial(jnp.dot, preferred_element_type=jnp.float32)
    v = dot(h4, wvt_ref[...]) + bv_ref[...]
    x_att = dot(v, wot_ref[...]) + bo_ref[...]
    w1 = jnp.maximum(dot(x_att, f0t_ref[...]) + fb0_ref[...], 0.0)
    w2 = jnp.maximum(dot(w1, f1t_ref[...]) + fb1_ref[...], 0.0)
    w_ref[...] = dot(w2, f2t_ref[...]) + fb2_ref[...]
    u = jnp.maximum(dot(x_att, v0t_ref[...]) + vb0_ref[...], 0.0)
    z = dot(u, v1t_ref[...]) + vb1_ref[...]
    vd_ref[...] = 1.0 / (1.0 + jnp.exp(-z))


def _full(shape):
    return pl.BlockSpec(shape, lambda i: tuple(0 for _ in shape))


_final_call = pl.pallas_call(
    _final_tc,
    grid=(G,),
    in_specs=[
        pl.BlockSpec((NC, B, D), lambda i: (0, i, 0)),
        pl.BlockSpec((B, D), lambda i: (i, 0)),
        pl.BlockSpec((B, 1), lambda i: (i, 0)),
        _full((1, D)),
        _full((D, D)),
        _full((1, D)),
        _full((D, D)),
        _full((1, D)),
        _full((D, 64)),
        _full((1, 64)),
        _full((64, 32)),
        _full((1, 32)),
        _full((32, 3)),
        _full((1, 3)),
        _full((D, 64)),
        _full((1, 64)),
        _full((64, 1)),
        _full((1, 1)),
    ],
    out_specs=[
        pl.BlockSpec((B, 3), lambda i: (i, 0)),
        pl.BlockSpec((B, 1), lambda i: (i, 0)),
    ],
    out_shape=[
        jax.ShapeDtypeStruct((N, 3), jnp.float32),
        jax.ShapeDtypeStruct((N, 1), jnp.float32),
    ],
)


def kernel(x, edge_index, batch, gcn_w0, gcn_b0, gcn_w1, gcn_b1, gcn_w2,
           gcn_b2, gcn_w3, gcn_b3, attn_in_w, attn_in_b, attn_out_w,
           attn_out_b, fc_w0, fc_b0, fc_w1, fc_b1, fc_w2, fc_b2, void_w0,
           void_b0, void_w1, void_b1):
    src2 = edge_index[0].reshape(E // CHUNK, CHUNK)
    dst2 = edge_index[1].reshape(E // CHUNK, CHUNK)
    z16 = jnp.zeros((N, 16), jnp.float32)
    z128 = jnp.zeros((N, D), jnp.float32)

    degtab = _deg_call()(dst2, z16)
    g, dinv = _init_call(degtab, x, gcn_w0.T)

    for wt, b in ((gcn_w1, gcn_b0), (gcn_w2, gcn_b1), (gcn_w3, gcn_b2)):
        acc = _edge_call()(g, src2, dst2, z128)
        g = _mid_call(acc, g, dinv, b[None], wt.T)

    acc = _edge_call()(g, src2, dst2, z128)
    wv_t = attn_in_w[2 * D:3 * D].T
    bv = attn_in_b[2 * D:][None]
    w_out, vd = _final_call(
        acc, g, dinv, gcn_b3[None], wv_t, bv, attn_out_w.T, attn_out_b[None],
        fc_w0.T, fc_b0[None], fc_w1.T, fc_b1[None], fc_w2.T, fc_b2[None],
        void_w0.T, void_b0[None], void_w1.T, void_b1[None])
    return (w_out, vd)


# speedup vs baseline: 21.8317x; 1.2625x over previous
import functools

import jax
import jax.numpy as jnp
from jax import lax
from jax.experimental import pallas as pl
from jax.experimental.pallas import tpu as pltpu
from jax.experimental.pallas import tpu_sc as plsc

N = 10000
D = 128
E = 320000
NC = 2
NS = 16
NW = NC * NS
EPT = E // NW
CHUNK = 125
NCH = EPT // CHUNK
SEG = 40
NSEG = NCH // SEG
RPT = 640
RPT_LAST = N - RPT * (NS - 1)


def _deg_body(dst_hbm, z16_hbm, out_hbm, idx_v, ones_v, tab_sh):
    c = lax.axis_index("c")
    s = lax.axis_index("s")
    one16 = jnp.ones((16,), jnp.float32)

    def fill_ones(i, carry):
        ones_v[i, :] = one16
        return carry

    lax.fori_loop(0, CHUNK, fill_ones, 0)

    @pl.when(s < NS - 1)
    def _():
        pltpu.sync_copy(z16_hbm.at[pl.ds(s * RPT, RPT)],
                        tab_sh.at[pl.ds(s * RPT, RPT)])

    @pl.when(s == NS - 1)
    def _():
        pltpu.sync_copy(z16_hbm.at[pl.ds(N - RPT_LAST, RPT_LAST)],
                        tab_sh.at[pl.ds(N - RPT_LAST, RPT_LAST)])

    plsc.subcore_barrier()

    tb = (c * NS + s) * NCH
    pltpu.sync_copy(dst_hbm.at[pl.ds(tb, NCH)], idx_v)

    def step(j, carry):
        pltpu.sync_copy(ones_v, tab_sh.at[idx_v.at[j]], add=True)
        return carry

    lax.fori_loop(0, NCH, step, 0)
    plsc.subcore_barrier()

    @pl.when(s < NS - 1)
    def _():
        pltpu.sync_copy(tab_sh.at[pl.ds(s * RPT, RPT)],
                        out_hbm.at[c, pl.ds(s * RPT, RPT)])

    @pl.when(s == NS - 1)
    def _():
        pltpu.sync_copy(tab_sh.at[pl.ds(N - RPT_LAST, RPT_LAST)],
                        out_hbm.at[c, pl.ds(N - RPT_LAST, RPT_LAST)])


@functools.cache
def _deg_call():
    mesh = plsc.VectorSubcoreMesh(
        core_axis_name="c", subcore_axis_name="s",
        num_cores=NC, num_subcores=NS)
    return pl.kernel(
        _deg_body,
        out_type=jax.ShapeDtypeStruct((NC, N, 16), jnp.float32),
        mesh=mesh,
        scratch_types=[
            pltpu.VMEM((NCH, CHUNK), jnp.int32),
            pltpu.VMEM((CHUNK, 16), jnp.float32),
            pltpu.VMEM_SHARED((N, 16), jnp.float32),
        ],
    )


def _edge_body(g_hbm, src_hbm, dst_hbm, z128_hbm, out_hbm,
               srcs_v, dsts_v, rows_a, rows_b, acc_sh, sem_a, sem_b):
    c = lax.axis_index("c")
    s = lax.axis_index("s")

    @pl.when(s < NS - 1)
    def _():
        pltpu.sync_copy(z128_hbm.at[pl.ds(s * RPT, RPT)],
                        acc_sh.at[pl.ds(s * RPT, RPT)])

    @pl.when(s == NS - 1)
    def _():
        pltpu.sync_copy(z128_hbm.at[pl.ds(N - RPT_LAST, RPT_LAST)],
                        acc_sh.at[pl.ds(N - RPT_LAST, RPT_LAST)])

    plsc.subcore_barrier()

    tb = (c * NS + s) * NCH

    def gather(j, buf, sem):
        pltpu.async_copy(g_hbm.at[srcs_v.at[j]], buf, sem)

    def gwait(buf, sem):
        pltpu.make_async_copy(g_hbm.at[srcs_v.at[0]], buf, sem).wait()

    def seg_loop(seg, carry):
        sb = tb + seg * SEG
        pltpu.sync_copy(src_hbm.at[pl.ds(sb, SEG)], srcs_v)
        pltpu.sync_copy(dst_hbm.at[pl.ds(sb, SEG)], dsts_v)
        gather(0, rows_a, sem_a)

        def step2(jj, carry2):
            j = 2 * jj
            gwait(rows_a, sem_a)
            gather(j + 1, rows_b, sem_b)
            pltpu.sync_copy(rows_a, acc_sh.at[dsts_v.at[j]], add=True)
            gwait(rows_b, sem_b)
            gather(jnp.minimum(j + 2, SEG - 1), rows_a, sem_a)
            pltpu.sync_copy(rows_b, acc_sh.at[dsts_v.at[j + 1]], add=True)
            return carry2

        lax.fori_loop(0, SEG // 2, step2, 0)
        gwait(rows_a, sem_a)
        return carry

    lax.fori_loop(0, NSEG, seg_loop, 0)
    plsc.subcore_barrier()

    @pl.when(s < NS - 1)
    def _():
        pltpu.sync_copy(acc_sh.at[pl.ds(s * RPT, RPT)],
                        out_hbm.at[c, pl.ds(s * RPT, RPT)])

    @pl.when(s == NS - 1)
    def _():
        pltpu.sync_copy(acc_sh.at[pl.ds(N - RPT_LAST, RPT_LAST)],
                        out_hbm.at[c, pl.ds(N - RPT_LAST, RPT_LAST)])


@functools.cache
def _edge_call():
    mesh = plsc.VectorSubcoreMesh(
        core_axis_name="c", subcore_axis_name="s",
        num_cores=NC, num_subcores=NS)
    return pl.kernel(
        _edge_body,
        out_type=jax.ShapeDtypeStruct((NC, N, D), jnp.float32),
        mesh=mesh,
        scratch_types=[
            pltpu.VMEM((SEG, CHUNK), jnp.int32),
            pltpu.VMEM((SEG, CHUNK), jnp.int32),
            pltpu.VMEM((CHUNK, D), jnp.float32),
            pltpu.VMEM((CHUNK, D), jnp.float32),
            pltpu.VMEM_SHARED((N, D), jnp.float32),
            pltpu.SemaphoreType.DMA,
            pltpu.SemaphoreType.DMA,
        ],
    )


B = 2000
G = N // B


def _init_tc(deg_ref, x_ref, w0t_ref, g0_ref, dinv_ref):
    deg = deg_ref[0, :, 0] + deg_ref[1, :, 0] + 1.0
    dinv = lax.rsqrt(deg)
    h = jnp.dot(x_ref[...], w0t_ref[...], preferred_element_type=jnp.float32)
    g0_ref[...] = h * dinv[:, None]
    dinv_ref[...] = dinv[:, None]


_init_call = pl.pallas_call(
    _init_tc,
    grid=(G,),
    in_specs=[
        pl.BlockSpec((NC, B, 16), lambda i: (0, i, 0)),
        pl.BlockSpec((B, D), lambda i: (i, 0)),
        pl.BlockSpec((D, D), lambda i: (0, 0)),
    ],
    out_specs=[
        pl.BlockSpec((B, D), lambda i: (i, 0)),
        pl.BlockSpec((B, 1), lambda i: (i, 0)),
    ],
    out_shape=[
        jax.ShapeDtypeStruct((N, D), jnp.float32),
        jax.ShapeDtypeStruct((N, 1), jnp.float32),
    ],
)


def _mid_tc(acc_ref, g_ref, dinv_ref, b_ref, wt_ref, out_ref):
    dinv = dinv_ref[...]
    t = (acc_ref[0] + acc_ref[1] + g_ref[...]) * dinv + b_ref[...]
    t = jnp.maximum(t, 0.0)
    out_ref[...] = jnp.dot(
        t, wt_ref[...], preferred_element_type=jnp.float32) * dinv


_mid_call = pl.pallas_call(
    _mid_tc,
    grid=(G,),
    in_specs=[
        pl.BlockSpec((NC, B, D), lambda i: (0, i, 0)),
        pl.BlockSpec((B, D), lambda i: (i, 0)),
        pl.BlockSpec((B, 1), lambda i: (i, 0)),
        pl.BlockSpec((1, D), lambda i: (0, 0)),
        pl.BlockSpec((D, D), lambda i: (0, 0)),
    ],
    out_specs=pl.BlockSpec((B, D), lambda i: (i, 0)),
    out_shape=jax.ShapeDtypeStruct((N, D), jnp.float32),
)


def _final_tc(acc_ref, g_ref, dinv_ref, b3_ref, wvt_ref, bv_ref, wot_ref,
              bo_ref, f0t_ref, fb0_ref, f1t_ref, fb1_ref, f2t_ref, fb2_ref,
              v0t_ref, vb0_ref, v1t_ref, vb1_ref, w_ref, vd_ref):
    dinv = dinv_ref[...]
    h4 = (acc_ref[0] + acc_ref[1] + g_ref[...]) * dinv + b3_ref[...]
    dot = functools.partial(jnp.dot, preferred_element_type=jnp.float32)
    v = dot(h4, wvt_ref[...]) + bv_ref[...]
    x_att = dot(v, wot_ref[...]) + bo_ref[...]
    w1 = jnp.maximum(dot(x_att, f0t_ref[...]) + fb0_ref[...], 0.0)
    w2 = jnp.maximum(dot(w1, f1t_ref[...]) + fb1_ref[...], 0.0)
    w_ref[...] = dot(w2, f2t_ref[...]) + fb2_ref[...]
    u = jnp.maximum(dot(x_att, v0t_ref[...]) + vb0_ref[...], 0.0)
    z = dot(u, v1t_ref[...]) + vb1_ref[...]
    vd_ref[...] = 1.0 / (1.0 + jnp.exp(-z))


def _full(shape):
    return pl.BlockSpec(shape, lambda i: tuple(0 for _ in shape))


_final_call = pl.pallas_call(
    _final_tc,
    grid=(G,),
    in_specs=[
        pl.BlockSpec((NC, B, D), lambda i: (0, i, 0)),
        pl.BlockSpec((B, D), lambda i: (i, 0)),
        pl.BlockSpec((B, 1), lambda i: (i, 0)),
        _full((1, D)),
        _full((D, D)),
        _full((1, D)),
        _full((D, D)),
        _full((1, D)),
        _full((D, 64)),
        _full((1, 64)),
        _full((64, 32)),
        _full((1, 32)),
        _full((32, 3)),
        _full((1, 3)),
        _full((D, 64)),
        _full((1, 64)),
        _full((64, 1)),
        _full((1, 1)),
    ],
    out_specs=[
        pl.BlockSpec((B, 3), lambda i: (i, 0)),
        pl.BlockSpec((B, 1), lambda i: (i, 0)),
    ],
    out_shape=[
        jax.ShapeDtypeStruct((N, 3), jnp.float32),
        jax.ShapeDtypeStruct((N, 1), jnp.float32),
    ],
)


def kernel(x, edge_index, batch, gcn_w0, gcn_b0, gcn_w1, gcn_b1, gcn_w2,
           gcn_b2, gcn_w3, gcn_b3, attn_in_w, attn_in_b, attn_out_w,
           attn_out_b, fc_w0, fc_b0, fc_w1, fc_b1, fc_w2, fc_b2, void_w0,
           void_b0, void_w1, void_b1):
    src2 = edge_index[0].reshape(E // CHUNK, CHUNK)
    dst2 = edge_index[1].reshape(E // CHUNK, CHUNK)
    z16 = jnp.zeros((N, 16), jnp.float32)
    z128 = jnp.zeros((N, D), jnp.float32)

    degtab = _deg_call()(dst2, z16)
    g, dinv = _init_call(degtab, x, gcn_w0.T)

    for wt, b in ((gcn_w1, gcn_b0), (gcn_w2, gcn_b1), (gcn_w3, gcn_b2)):
        acc = _edge_call()(g, src2, dst2, z128)
        g = _mid_call(acc, g, dinv, b[None], wt.T)

    acc = _edge_call()(g, src2, dst2, z128)
    wv_t = attn_in_w[2 * D:3 * D].T
    bv = attn_in_b[2 * D:][None]
    w_out, vd = _final_call(
        acc, g, dinv, gcn_b3[None], wv_t, bv, attn_out_w.T, attn_out_b[None],
        fc_w0.T, fc_b0[None], fc_w1.T, fc_b1[None], fc_w2.T, fc_b2[None],
        void_w0.T, void_b0[None], void_w1.T, void_b1[None])
    return (w_out, vd)
